# baseline (speedup 1.0000x reference)
import functools

import jax
import jax.numpy as jnp
from jax import lax
from jax.experimental import pallas as pl
from jax.experimental.pallas import tpu as pltpu
from jax.experimental.pallas import tpu_sc as plsc

_BATCH = 8
_NUM_ROIS = 100
_RES = 28
_NUM_CLASSES = 91
_ROWS = _BATCH * _NUM_ROIS
_D = _RES * _RES
_ROWS_PER_W = 32
_ACTIVE_W = _ROWS // _ROWS_PER_W
_NC = 2
_L = 16


@functools.partial(
    pl.kernel,
    mesh=plsc.VectorSubcoreMesh(core_axis_name="c", subcore_axis_name="s"),
    out_type=jax.ShapeDtypeStruct((_ROWS, _D), jnp.float32),
    scratch_types=[
        pltpu.VMEM((_ROWS_PER_W,), jnp.int32),
        pltpu.VMEM((_ROWS_PER_W, _D), jnp.float32),
        pltpu.SemaphoreType.DMA,
    ],
    compiler_params=pltpu.CompilerParams(use_tc_tiling_on_sc=False),
)
def _sc_gather(table_hbm, cls_hbm, out_hbm, idx_v, rows_v, sem):
    wid = lax.axis_index("s") * _NC + lax.axis_index("c")

    @pl.when(wid < _ACTIVE_W)
    def _():
        base = wid * _ROWS_PER_W
        pltpu.sync_copy(cls_hbm.at[pl.ds(base, _ROWS_PER_W)], idx_v)
        for c in range(_ROWS_PER_W // _L):
            cls_vec = idx_v[pl.ds(c * _L, _L)]
            row_ids = base + c * _L + lax.iota(jnp.int32, _L)
            idx_v[pl.ds(c * _L, _L)] = row_ids * _NUM_CLASSES + cls_vec
        pltpu.async_copy(table_hbm.at[idx_v], rows_v, sem).wait()
        pltpu.sync_copy(rows_v, out_hbm.at[pl.ds(base, _ROWS_PER_W)])


def kernel(mask_outputs, class_indices):
    table = mask_outputs.reshape(_ROWS * _NUM_CLASSES, _D)
    cls = class_indices.reshape(_ROWS).astype(jnp.int32)
    out = _sc_gather(table, cls)
    return out.reshape(_BATCH, _NUM_ROIS, _RES, _RES)


# speedup vs baseline: 2.5588x; 2.5588x over previous
import functools

import jax
import jax.numpy as jnp
from jax import lax
from jax.experimental import pallas as pl
from jax.experimental.pallas import tpu as pltpu
from jax.experimental.pallas import tpu_sc as plsc

_BATCH = 8
_NUM_ROIS = 100
_RES = 28
_NUM_CLASSES = 91
_ROWS = _BATCH * _NUM_ROIS
_ROWS_PER_W = 32
_ACTIVE_W = _ROWS // _ROWS_PER_W
_NC = 2


@functools.partial(
    pl.kernel,
    mesh=plsc.VectorSubcoreMesh(core_axis_name="c", subcore_axis_name="s"),
    out_type=jax.ShapeDtypeStruct((_ROWS, _RES, _RES), jnp.float32),
    scratch_types=[
        pltpu.VMEM((_ROWS_PER_W,), jnp.int32),
        pltpu.SemaphoreType.DMA,
    ],
)
def _sc_gather(masks_hbm, cls_hbm, out_hbm, cls_v, sem):
    wid = lax.axis_index("s") * _NC + lax.axis_index("c")

    @pl.when(wid < _ACTIVE_W)
    def _():
        base = wid * _ROWS_PER_W
        pltpu.sync_copy(cls_hbm.at[pl.ds(base, _ROWS_PER_W)], cls_v)
        copies = []
        for j in range(_ROWS_PER_W):
            if j % 16 == 0:
                chunk = cls_v[pl.ds(j, 16)]
            c = chunk[j % 16]
            copies.append(pltpu.make_async_copy(
                masks_hbm.at[base + j, c], out_hbm.at[base + j], sem))
        for cp in copies:
            cp.start()
        for cp in copies:
            cp.wait()


def kernel(mask_outputs, class_indices):
    masks = mask_outputs.reshape(_ROWS, _NUM_CLASSES, _RES, _RES)
    cls = class_indices.reshape(_ROWS).astype(jnp.int32)
    out = _sc_gather(masks, cls)
    return out.reshape(_BATCH, _NUM_ROIS, _RES, _RES)
